# baseline (speedup 1.0000x reference)
import functools

import numpy as np
import jax
import jax.numpy as jnp
from jax.experimental import pallas as pl
from jax.experimental.pallas import tpu as pltpu

_D = 18
_GS = 9
_NC = 512
_NOUT = 2 * _NC + 3
_SQRT_D = np.float32(np.sqrt(np.float32(18.0)))
_QS = np.float32(np.float32(1.0) / _SQRT_D)
_HALF_QS = np.float32(_QS / np.float32(2.0))
_ROWS = 512


def _weights():
    codes = np.arange(_NC)
    gb = 2 ** np.arange(_GS - 1, -1, -1)
    cb = (((codes[:, None] // gb) % 2) * 2 - 1).astype(np.float32)
    w = np.zeros((3 * _D, _NOUT), np.float32)
    w[:_GS, :_NC] = cb.T
    w[_GS:_D, _NC:2 * _NC] = cb.T
    w[_D:2 * _D] = w[:_D]
    basis = (2.0 ** np.arange(_D - 1, -1, -1)).astype(np.float32)
    w[2 * _D:, 2 * _NC] = basis
    w[2 * _D:2 * _D + _GS, 2 * _NC + 1] = basis[_GS:]
    w[2 * _D + _GS:, 2 * _NC + 2] = basis[_GS:]
    return jnp.asarray(w)


def _group_ones():
    m = np.zeros((2, 2 * _NC), np.float32)
    m[0, :_NC] = 1.0
    m[1, _NC:] = 1.0
    return jnp.asarray(m)


def _bsq_kernel(z_ref, w_ref, ones_ref, zq_ref, idx_ref, gidx_ref, avgp_ref,
                loss_ref, cbe_ref, acc_ref, s_ref, *, ntot):
    pid = pl.program_id(0)
    nsteps = pl.num_programs(0)

    @pl.when(pid == 0)
    def _init():
        acc_ref[...] = jnp.zeros_like(acc_ref)
        s_ref[0] = jnp.float32(0.0)
        s_ref[1] = jnp.float32(0.0)

    z = z_ref[...]
    zhat = jnp.where(z > 0, jnp.float32(1.0), jnp.float32(-1.0))
    zq = zhat * _QS
    zq_ref[...] = zq

    cz = z * jnp.float32(2.0 * float(_QS))
    hi = cz.astype(jnp.bfloat16).astype(jnp.float32)
    lo = cz - hi
    x = jnp.concatenate([hi, lo, zhat], axis=1)
    o1 = jnp.dot(x, w_ref[...], preferred_element_type=jnp.float32)

    idx_f = 131071.5 + _HALF_QS * o1[:, 2 * _NC:2 * _NC + 1]
    idx_ref[...] = idx_f.astype(jnp.int32)
    g = 255.5 + _HALF_QS * o1[:, 2 * _NC + 1:]
    gidx_ref[...] = g.astype(jnp.int32)

    e = jnp.exp(o1[:, :2 * _NC])
    s2 = jax.lax.dot_general(ones_ref[...], e, (((1,), (1,)), ((), ())),
                             preferred_element_type=jnp.float32)
    r2 = 1.0 / s2
    res = jax.lax.dot_general(r2, e, (((1,), (0,)), ((), ())),
                              preferred_element_type=jnp.float32)
    acc_ref[...] += res

    p = jax.nn.sigmoid(z * jnp.float32(-4.0 * float(_QS)))
    ent = -(p * jnp.log(p + 1e-8) + (1.0 - p) * jnp.log((1.0 - p) + 1e-8))
    s_ref[0] += jnp.sum(ent)
    diff = zq - z
    s_ref[1] += jnp.sum(diff * diff)

    @pl.when(pid == nsteps - 1)
    def _fin():
        inv_n = jnp.float32(1.0 / ntot)
        acc = acc_ref[...] * inv_n
        avgp = jnp.concatenate([acc[0:1, :_NC], acc[1:2, _NC:]], axis=0)
        avgp_ref[...] = avgp
        cbe = -jnp.sum(avgp * jnp.log(avgp + 1e-8))
        cbe_ref[...] = jnp.reshape(cbe, (1, 1))
        pse = s_ref[0] * inv_n
        commit = 0.25 * (s_ref[1] * inv_n)
        loss_ref[...] = jnp.reshape(commit + pse - cbe, (1, 1))


def kernel(z):
    b, s, d = z.shape
    n = b * s
    zf = z.reshape(n, d)
    w = _weights()
    ones_bd = _group_ones()
    grid = n // _ROWS
    outs = pl.pallas_call(
        functools.partial(_bsq_kernel, ntot=float(n)),
        grid=(grid,),
        in_specs=[
            pl.BlockSpec((_ROWS, d), lambda i: (i, 0)),
            pl.BlockSpec((3 * _D, _NOUT), lambda i: (0, 0)),
            pl.BlockSpec((2, 2 * _NC), lambda i: (0, 0)),
        ],
        out_specs=[
            pl.BlockSpec((_ROWS, d), lambda i: (i, 0)),
            pl.BlockSpec((_ROWS, 1), lambda i: (i, 0)),
            pl.BlockSpec((_ROWS, 2), lambda i: (i, 0)),
            pl.BlockSpec((2, _NC), lambda i: (0, 0)),
            pl.BlockSpec((1, 1), lambda i: (0, 0)),
            pl.BlockSpec((1, 1), lambda i: (0, 0)),
        ],
        out_shape=[
            jax.ShapeDtypeStruct((n, d), jnp.float32),
            jax.ShapeDtypeStruct((n, 1), jnp.int32),
            jax.ShapeDtypeStruct((n, 2), jnp.int32),
            jax.ShapeDtypeStruct((2, _NC), jnp.float32),
            jax.ShapeDtypeStruct((1, 1), jnp.float32),
            jax.ShapeDtypeStruct((1, 1), jnp.float32),
        ],
        scratch_shapes=[
            pltpu.VMEM((2, 2 * _NC), jnp.float32),
            pltpu.SMEM((2,), jnp.float32),
        ],
        compiler_params=pltpu.CompilerParams(
            dimension_semantics=("arbitrary",)),
    )(zf, w, ones_bd)
    zq, idx, gidx, avgp, loss, cbe = outs
    zq = zq.reshape(b, s, d)
    indices = idx.reshape(b, s).astype(jnp.int64)
    group_indices = gidx.reshape(b, s, 2).astype(jnp.int64)
    return (zq, loss[0, 0], cbe[0, 0], indices, group_indices, avgp)


# speedup vs baseline: 1.0868x; 1.0868x over previous
import functools

import numpy as np
import jax
import jax.numpy as jnp
from jax.experimental import pallas as pl
from jax.experimental.pallas import tpu as pltpu

_D = 18
_GS = 9
_NC = 512
_NOUT = 2 * _NC + 3
_SQRT_D = np.float32(np.sqrt(np.float32(18.0)))
_QS = np.float32(np.float32(1.0) / _SQRT_D)
_HALF_QS = np.float32(_QS / np.float32(2.0))
_ROWS = 1024


def _weights():
    codes = np.arange(_NC)
    gb = 2 ** np.arange(_GS - 1, -1, -1)
    cb = (((codes[:, None] // gb) % 2) * 2 - 1).astype(np.float32)
    w = np.zeros((3 * _D, _NOUT), np.float32)
    w[:_GS, :_NC] = cb.T
    w[_GS:_D, _NC:2 * _NC] = cb.T
    w[_D:2 * _D] = w[:_D]
    basis = (2.0 ** np.arange(_D - 1, -1, -1)).astype(np.float32)
    w[2 * _D:, 2 * _NC] = basis
    w[2 * _D:2 * _D + _GS, 2 * _NC + 1] = basis[_GS:]
    w[2 * _D + _GS:, 2 * _NC + 2] = basis[_GS:]
    return jnp.asarray(w)


def _group_ones():
    m = np.zeros((2, 2 * _NC), np.float32)
    m[0, :_NC] = 1.0
    m[1, _NC:] = 1.0
    return jnp.asarray(m)


def _bsq_kernel(z_ref, w_ref, ones_ref, zq_ref, idx_ref, gidx_ref, avgp_ref,
                loss_ref, cbe_ref, acc_ref, s_ref, *, ntot):
    pid = pl.program_id(0)
    nsteps = pl.num_programs(0)

    @pl.when(pid == 0)
    def _init():
        acc_ref[...] = jnp.zeros_like(acc_ref)
        s_ref[0] = jnp.float32(0.0)
        s_ref[1] = jnp.float32(0.0)

    z = z_ref[...]
    zhat = jnp.where(z > 0, jnp.float32(1.0), jnp.float32(-1.0))
    zq = zhat * _QS
    zq_ref[...] = zq

    cz = z * jnp.float32(2.0 * float(_QS))
    hi = cz.astype(jnp.bfloat16).astype(jnp.float32)
    lo = cz - hi
    x = jnp.concatenate([hi, lo, zhat], axis=1)
    o1 = jnp.dot(x, w_ref[...], preferred_element_type=jnp.float32)

    idx_f = 131071.5 + _HALF_QS * o1[:, 2 * _NC:2 * _NC + 1]
    idx_ref[...] = idx_f.astype(jnp.int32)
    g = 255.5 + _HALF_QS * o1[:, 2 * _NC + 1:]
    gidx_ref[...] = g.astype(jnp.int32)

    e = jnp.exp(o1[:, :2 * _NC])
    s2 = jax.lax.dot_general(ones_ref[...], e, (((1,), (1,)), ((), ())),
                             preferred_element_type=jnp.float32)
    r2 = 1.0 / s2
    res = jax.lax.dot_general(r2, e, (((1,), (0,)), ((), ())),
                              preferred_element_type=jnp.float32)
    acc_ref[...] += res

    p = jax.nn.sigmoid(z * jnp.float32(-4.0 * float(_QS)))
    ent = -(p * jnp.log(p + 1e-8) + (1.0 - p) * jnp.log((1.0 - p) + 1e-8))
    s_ref[0] += jnp.sum(ent)
    diff = zq - z
    s_ref[1] += jnp.sum(diff * diff)

    @pl.when(pid == nsteps - 1)
    def _fin():
        inv_n = jnp.float32(1.0 / ntot)
        acc = acc_ref[...] * inv_n
        avgp = jnp.concatenate([acc[0:1, :_NC], acc[1:2, _NC:]], axis=0)
        avgp_ref[...] = avgp
        cbe = -jnp.sum(avgp * jnp.log(avgp + 1e-8))
        cbe_ref[...] = jnp.reshape(cbe, (1, 1))
        pse = s_ref[0] * inv_n
        commit = 0.25 * (s_ref[1] * inv_n)
        loss_ref[...] = jnp.reshape(commit + pse - cbe, (1, 1))


def kernel(z):
    b, s, d = z.shape
    n = b * s
    zf = z.reshape(n, d)
    w = _weights()
    ones_bd = _group_ones()
    grid = n // _ROWS
    outs = pl.pallas_call(
        functools.partial(_bsq_kernel, ntot=float(n)),
        grid=(grid,),
        in_specs=[
            pl.BlockSpec((_ROWS, d), lambda i: (i, 0)),
            pl.BlockSpec((3 * _D, _NOUT), lambda i: (0, 0)),
            pl.BlockSpec((2, 2 * _NC), lambda i: (0, 0)),
        ],
        out_specs=[
            pl.BlockSpec((_ROWS, d), lambda i: (i, 0)),
            pl.BlockSpec((_ROWS, 1), lambda i: (i, 0)),
            pl.BlockSpec((_ROWS, 2), lambda i: (i, 0)),
            pl.BlockSpec((2, _NC), lambda i: (0, 0)),
            pl.BlockSpec((1, 1), lambda i: (0, 0)),
            pl.BlockSpec((1, 1), lambda i: (0, 0)),
        ],
        out_shape=[
            jax.ShapeDtypeStruct((n, d), jnp.float32),
            jax.ShapeDtypeStruct((n, 1), jnp.int32),
            jax.ShapeDtypeStruct((n, 2), jnp.int32),
            jax.ShapeDtypeStruct((2, _NC), jnp.float32),
            jax.ShapeDtypeStruct((1, 1), jnp.float32),
            jax.ShapeDtypeStruct((1, 1), jnp.float32),
        ],
        scratch_shapes=[
            pltpu.VMEM((2, 2 * _NC), jnp.float32),
            pltpu.SMEM((2,), jnp.float32),
        ],
        compiler_params=pltpu.CompilerParams(
            dimension_semantics=("arbitrary",)),
    )(zf, w, ones_bd)
    zq, idx, gidx, avgp, loss, cbe = outs
    zq = zq.reshape(b, s, d)
    indices = idx.reshape(b, s).astype(jnp.int64)
    group_indices = gidx.reshape(b, s, 2).astype(jnp.int64)
    return (zq, loss[0, 0], cbe[0, 0], indices, group_indices, avgp)
